# baseline (speedup 1.0000x reference)
import functools

import jax
import jax.numpy as jnp
from jax import lax
from jax.experimental import pallas as pl
from jax.experimental.pallas import tpu as pltpu
from jax.experimental.pallas import tpu_sc as plsc

N = 10000
D = 128
E = 320000
K_HOPS = 3

NC = 2
NS = 16
NW = NC * NS

EPW = E // NW
CHUNK = 80
NCHUNK = EPW // CHUNK
IBLK = 5
NIBLK = NCHUNK // IBLK
NBUF = 4

ACC0 = 624
ACC15 = 640

NP = 10240
DEGW = NP // NW

ACC_ROWS = NP // NS
ZROWS = 80


def _mesh():
  return plsc.VectorSubcoreMesh(
      core_axis_name="c", subcore_axis_name="s", num_cores=NC, num_subcores=NS
  )


def _zero_fill(vref, nwords):
  z = jnp.zeros((16,), jnp.float32)

  def body(i, carry):
    vref[pl.ds(i * 16, 16)] = z
    return carry

  lax.fori_loop(0, nwords // 16, body, 0)


def _zero_fill_2d(vref, nrows):
  z = jnp.zeros((16,), jnp.float32)

  def body(r, carry):
    for c in range(D // 16):
      vref[r, pl.ds(c * 16, 16)] = z
    return carry

  lax.fori_loop(0, nrows, body, 0)


def _deg_body(dst_hbm, degp_hbm, idx_v, ones_v, zbuf_v, deg_sh):
  sid = lax.axis_index("s")
  cid = lax.axis_index("c")
  wid = cid * NS + sid

  zslice = NP // NS
  _zero_fill(zbuf_v, zslice)
  pltpu.sync_copy(zbuf_v, deg_sh.at[pl.ds(sid * zslice, zslice)])

  one = jnp.ones((16,), jnp.float32)
  for i in range(8):
    ones_v[pl.ds(i * 16, 16)] = one

  pltpu.sync_copy(dst_hbm.at[wid], idx_v)
  plsc.subcore_barrier()

  def step(j, carry):
    pltpu.sync_copy(ones_v.at[pl.ds(0, CHUNK)], deg_sh.at[idx_v.at[j]],
                    add=True)
    return carry

  lax.fori_loop(0, NCHUNK, step, 0)
  plsc.subcore_barrier()

  pltpu.sync_copy(
      deg_sh.at[pl.ds(sid * zslice, zslice)],
      degp_hbm.at[pl.ds(cid * NP + sid * zslice, zslice)],
  )


def _k_deg(dst3):
  f = pl.kernel(
      _deg_body,
      out_type=jax.ShapeDtypeStruct((NC * NP,), jnp.float32),
      mesh=_mesh(),
      scratch_types=[
          pltpu.VMEM((NCHUNK, CHUNK), jnp.int32),
          pltpu.VMEM((128,), jnp.float32),
          pltpu.VMEM((NP // NS,), jnp.float32),
          pltpu.VMEM_SHARED((NP,), jnp.float32),
      ],
  )
  return f(dst3)


def _hop_body(g_hbm, src_hbm, dst_hbm, p_hbm,
              src_v, dst_v, rows_v, gsem, ssem, acc_sh):
  sid = lax.axis_index("s")
  cid = lax.axis_index("c")
  wid = cid * NS + sid

  _zero_fill_2d(rows_v.at[0], ZROWS)

  @pl.when(sid < NS - 1)
  def _zero_624():
    for z in range(7):
      pltpu.sync_copy(
          rows_v.at[0, pl.ds(0, ZROWS)],
          acc_sh.at[pl.ds(sid * ACC0 + z * ZROWS, ZROWS)],
      )
    pltpu.sync_copy(
        rows_v.at[0, pl.ds(0, ACC0 - 7 * ZROWS)],
        acc_sh.at[pl.ds(sid * ACC0 + 7 * ZROWS, ACC0 - 7 * ZROWS)],
    )

  @pl.when(sid == NS - 1)
  def _zero_640():
    for z in range(8):
      pltpu.sync_copy(
          rows_v.at[0, pl.ds(0, ZROWS)],
          acc_sh.at[pl.ds((NS - 1) * ACC0 + z * ZROWS, ZROWS)],
      )

  def sref(j):
    return src_v.at[(j // IBLK) % 2, j % IBLK]

  def dref(j):
    return dst_v.at[(j // IBLK) % 2, j % IBLK]

  def load_iblk(b):
    pltpu.sync_copy(src_hbm.at[wid, b], src_v.at[b % 2])
    pltpu.sync_copy(dst_hbm.at[wid, b], dst_v.at[b % 2])

  load_iblk(0)
  plsc.subcore_barrier()

  pltpu.async_copy(g_hbm.at[sref(0)], rows_v.at[0], gsem)
  pltpu.async_copy(g_hbm.at[sref(1)], rows_v.at[1], gsem)
  pltpu.async_copy(g_hbm.at[sref(2)], rows_v.at[2], gsem)

  def step(j, carry):
    buf = j % NBUF

    @pl.when(((j + 3) % IBLK == 0) & (j + 3 < NCHUNK))
    def _load_next_iblk():
      load_iblk((j + 3) // IBLK)

    pltpu.make_async_copy(g_hbm.at[sref(j)], rows_v.at[buf], gsem).wait()
    pltpu.async_copy(rows_v.at[buf], acc_sh.at[dref(j)], ssem, add=True)

    @pl.when(j > 0)
    def _drain_prev_scatter():
      pltpu.make_async_copy(
          rows_v.at[(j - 1) % NBUF], acc_sh.at[dref(j - 1)], ssem
      ).wait()

    @pl.when(j + 3 < NCHUNK)
    def _issue_next_gather():
      pltpu.async_copy(g_hbm.at[sref(j + 3)], rows_v.at[(j + 3) % NBUF], gsem)

    return carry

  lax.fori_loop(0, NCHUNK, step, 0)
  pltpu.make_async_copy(
      rows_v.at[(NCHUNK - 1) % NBUF], acc_sh.at[dref(NCHUNK - 1)], ssem
  ).wait()
  plsc.subcore_barrier()

  @pl.when(sid < NS - 1)
  def _write_624():
    pltpu.sync_copy(
        acc_sh.at[pl.ds(sid * ACC0, ACC0)],
        p_hbm.at[cid, pl.ds(sid * ACC0, ACC0)],
    )

  @pl.when(sid == NS - 1)
  def _write_640():
    pltpu.sync_copy(
        acc_sh.at[pl.ds((NS - 1) * ACC0, ACC15)],
        p_hbm.at[cid, pl.ds((NS - 1) * ACC0, ACC15)],
    )


def _k_hop(g, src3, dst3):
  f = pl.kernel(
      _hop_body,
      out_type=jax.ShapeDtypeStruct((NC, N, D), jnp.float32),
      mesh=_mesh(),
      scratch_types=[
          pltpu.VMEM((2, IBLK, CHUNK), jnp.int32),
          pltpu.VMEM((2, IBLK, CHUNK), jnp.int32),
          pltpu.VMEM((NBUF, CHUNK, D), jnp.float32),
          pltpu.SemaphoreType.DMA,
          pltpu.SemaphoreType.DMA,
          pltpu.VMEM_SHARED((N, D), jnp.float32),
      ],
  )
  return f(g, src3, dst3)


def _merge_body(final, p_ref, g_ref, s_ref, b_ref, out_ref):
  v = (p_ref[0] + p_ref[1] + g_ref[...]) * s_ref[:N][:, None]
  if final:
    v = v + b_ref[...][None, :]
  out_ref[...] = v


def _k_merge(final, p, g, scale, b):
  return pl.pallas_call(
      functools.partial(_merge_body, final),
      out_shape=jax.ShapeDtypeStruct((N, D), jnp.float32),
  )(p, g, scale, b)


def _pre_body(x_ref, w_ref, degp_ref, g0_ref, d_ref, d2_ref):
  deg = degp_ref[0, :] + degp_ref[1, :] + 1.0
  d = lax.rsqrt(deg)
  d_ref[...] = d
  d2_ref[...] = 1.0 / deg
  y = lax.dot_general(
      x_ref[...], w_ref[...], (((1,), (1,)), ((), ())),
      preferred_element_type=jnp.float32,
  )
  g0_ref[...] = y * d[:N, None]


def _k_pre(x, W, degp):
  return pl.pallas_call(
      _pre_body,
      out_shape=(
          jax.ShapeDtypeStruct((N, D), jnp.float32),
          jax.ShapeDtypeStruct((NP,), jnp.float32),
          jax.ShapeDtypeStruct((NP,), jnp.float32),
      ),
  )(x, W, degp.reshape(NC, NP))


def kernel(x, edge_index, W, b):
  src = edge_index[0].astype(jnp.int32).reshape(NW, NIBLK, IBLK, CHUNK)
  dst = edge_index[1].astype(jnp.int32).reshape(NW, NIBLK, IBLK, CHUNK)

  degp = _k_deg(edge_index[1].astype(jnp.int32).reshape(NW, NCHUNK, CHUNK))
  g, d, d2 = _k_pre(x, W, degp)

  for k in range(K_HOPS):
    p = _k_hop(g, src, dst)
    final = k == K_HOPS - 1
    g = _k_merge(final, p, g, d if final else d2, b)

  return g


# speedup vs baseline: 1.0509x; 1.0509x over previous
import functools

import jax
import jax.numpy as jnp
from jax import lax
from jax.experimental import pallas as pl
from jax.experimental.pallas import tpu as pltpu
from jax.experimental.pallas import tpu_sc as plsc

N = 10000
D = 128
E = 320000
K_HOPS = 3

NC = 2
NS = 16
NW = NC * NS

EPW = E // NW
CHUNK = 100
NCHUNK = EPW // CHUNK
IBLK = 10
NIBLK = NCHUNK // IBLK
NBUF = 3

ACC0 = 624
ACC15 = 640

NP = 10240
DEGW = NP // NW

ACC_ROWS = NP // NS
ZROWS = 80


def _mesh():
  return plsc.VectorSubcoreMesh(
      core_axis_name="c", subcore_axis_name="s", num_cores=NC, num_subcores=NS
  )


def _zero_fill(vref, nwords):
  z = jnp.zeros((16,), jnp.float32)

  def body(i, carry):
    vref[pl.ds(i * 16, 16)] = z
    return carry

  lax.fori_loop(0, nwords // 16, body, 0)


def _zero_fill_2d(vref, nrows):
  z = jnp.zeros((16,), jnp.float32)

  def body(r, carry):
    for c in range(D // 16):
      vref[r, pl.ds(c * 16, 16)] = z
    return carry

  lax.fori_loop(0, nrows, body, 0)


def _deg_body(dst_hbm, degp_hbm, idx_v, ones_v, zbuf_v, deg_sh):
  sid = lax.axis_index("s")
  cid = lax.axis_index("c")
  wid = cid * NS + sid

  zslice = NP // NS
  _zero_fill(zbuf_v, zslice)
  pltpu.sync_copy(zbuf_v, deg_sh.at[pl.ds(sid * zslice, zslice)])

  one = jnp.ones((16,), jnp.float32)
  for i in range(8):
    ones_v[pl.ds(i * 16, 16)] = one

  pltpu.sync_copy(dst_hbm.at[wid], idx_v)
  plsc.subcore_barrier()

  def step(j, carry):
    pltpu.sync_copy(ones_v.at[pl.ds(0, CHUNK)], deg_sh.at[idx_v.at[j]],
                    add=True)
    return carry

  lax.fori_loop(0, NCHUNK, step, 0)
  plsc.subcore_barrier()

  pltpu.sync_copy(
      deg_sh.at[pl.ds(sid * zslice, zslice)],
      degp_hbm.at[pl.ds(cid * NP + sid * zslice, zslice)],
  )


def _k_deg(dst3):
  f = pl.kernel(
      _deg_body,
      out_type=jax.ShapeDtypeStruct((NC * NP,), jnp.float32),
      mesh=_mesh(),
      scratch_types=[
          pltpu.VMEM((NCHUNK, CHUNK), jnp.int32),
          pltpu.VMEM((128,), jnp.float32),
          pltpu.VMEM((NP // NS,), jnp.float32),
          pltpu.VMEM_SHARED((NP,), jnp.float32),
      ],
  )
  return f(dst3)


def _hop_body(g_hbm, src_hbm, dst_hbm, p_hbm,
              src_v, dst_v, rows_v, gsem, ssem, acc_sh):
  sid = lax.axis_index("s")
  cid = lax.axis_index("c")
  wid = cid * NS + sid

  _zero_fill_2d(rows_v.at[0], ZROWS)

  @pl.when(sid < NS - 1)
  def _zero_624():
    for z in range(7):
      pltpu.sync_copy(
          rows_v.at[0, pl.ds(0, ZROWS)],
          acc_sh.at[pl.ds(sid * ACC0 + z * ZROWS, ZROWS)],
      )
    pltpu.sync_copy(
        rows_v.at[0, pl.ds(0, ACC0 - 7 * ZROWS)],
        acc_sh.at[pl.ds(sid * ACC0 + 7 * ZROWS, ACC0 - 7 * ZROWS)],
    )

  @pl.when(sid == NS - 1)
  def _zero_640():
    for z in range(8):
      pltpu.sync_copy(
          rows_v.at[0, pl.ds(0, ZROWS)],
          acc_sh.at[pl.ds((NS - 1) * ACC0 + z * ZROWS, ZROWS)],
      )

  def sref(j):
    return src_v.at[(j // IBLK) % 2, j % IBLK]

  def dref(j):
    return dst_v.at[(j // IBLK) % 2, j % IBLK]

  def load_iblk(b):
    pltpu.sync_copy(src_hbm.at[wid, b], src_v.at[b % 2])
    pltpu.sync_copy(dst_hbm.at[wid, b], dst_v.at[b % 2])

  load_iblk(0)
  plsc.subcore_barrier()

  pltpu.async_copy(g_hbm.at[sref(0)], rows_v.at[0], gsem)
  pltpu.async_copy(g_hbm.at[sref(1)], rows_v.at[1], gsem)

  def step(j, carry):
    buf = j % NBUF

    @pl.when(((j + 2) % IBLK == 0) & (j + 2 < NCHUNK))
    def _load_next_iblk():
      load_iblk((j + 2) // IBLK)

    pltpu.make_async_copy(g_hbm.at[sref(j)], rows_v.at[buf], gsem).wait()
    pltpu.async_copy(rows_v.at[buf], acc_sh.at[dref(j)], ssem, add=True)

    @pl.when(j > 0)
    def _drain_prev_scatter():
      pltpu.make_async_copy(
          rows_v.at[(j - 1) % NBUF], acc_sh.at[dref(j - 1)], ssem
      ).wait()

    @pl.when(j + 2 < NCHUNK)
    def _issue_next_gather():
      pltpu.async_copy(g_hbm.at[sref(j + 2)], rows_v.at[(j + 2) % NBUF], gsem)

    return carry

  lax.fori_loop(0, NCHUNK, step, 0)
  pltpu.make_async_copy(
      rows_v.at[(NCHUNK - 1) % NBUF], acc_sh.at[dref(NCHUNK - 1)], ssem
  ).wait()
  plsc.subcore_barrier()

  @pl.when(sid < NS - 1)
  def _write_624():
    pltpu.sync_copy(
        acc_sh.at[pl.ds(sid * ACC0, ACC0)],
        p_hbm.at[cid, pl.ds(sid * ACC0, ACC0)],
    )

  @pl.when(sid == NS - 1)
  def _write_640():
    pltpu.sync_copy(
        acc_sh.at[pl.ds((NS - 1) * ACC0, ACC15)],
        p_hbm.at[cid, pl.ds((NS - 1) * ACC0, ACC15)],
    )


def _k_hop(g, src3, dst3):
  f = pl.kernel(
      _hop_body,
      out_type=jax.ShapeDtypeStruct((NC, N, D), jnp.float32),
      mesh=_mesh(),
      scratch_types=[
          pltpu.VMEM((2, IBLK, CHUNK), jnp.int32),
          pltpu.VMEM((2, IBLK, CHUNK), jnp.int32),
          pltpu.VMEM((NBUF, CHUNK, D), jnp.float32),
          pltpu.SemaphoreType.DMA,
          pltpu.SemaphoreType.DMA,
          pltpu.VMEM_SHARED((N, D), jnp.float32),
      ],
  )
  return f(g, src3, dst3)


def _merge_body(final, p_ref, g_ref, s_ref, b_ref, out_ref):
  v = (p_ref[0] + p_ref[1] + g_ref[...]) * s_ref[:N][:, None]
  if final:
    v = v + b_ref[...][None, :]
  out_ref[...] = v


def _k_merge(final, p, g, scale, b):
  return pl.pallas_call(
      functools.partial(_merge_body, final),
      out_shape=jax.ShapeDtypeStruct((N, D), jnp.float32),
  )(p, g, scale, b)


def _pre_body(x_ref, w_ref, degp_ref, g0_ref, d_ref, d2_ref):
  deg = degp_ref[0, :] + degp_ref[1, :] + 1.0
  d = lax.rsqrt(deg)
  d_ref[...] = d
  d2_ref[...] = 1.0 / deg
  y = lax.dot_general(
      x_ref[...], w_ref[...], (((1,), (1,)), ((), ())),
      preferred_element_type=jnp.float32,
  )
  g0_ref[...] = y * d[:N, None]


def _k_pre(x, W, degp):
  return pl.pallas_call(
      _pre_body,
      out_shape=(
          jax.ShapeDtypeStruct((N, D), jnp.float32),
          jax.ShapeDtypeStruct((NP,), jnp.float32),
          jax.ShapeDtypeStruct((NP,), jnp.float32),
      ),
  )(x, W, degp.reshape(NC, NP))


def kernel(x, edge_index, W, b):
  src = edge_index[0].astype(jnp.int32).reshape(NW, NIBLK, IBLK, CHUNK)
  dst = edge_index[1].astype(jnp.int32).reshape(NW, NIBLK, IBLK, CHUNK)

  degp = _k_deg(edge_index[1].astype(jnp.int32).reshape(NW, NCHUNK, CHUNK))
  g, d, d2 = _k_pre(x, W, degp)

  for k in range(K_HOPS):
    p = _k_hop(g, src, dst)
    final = k == K_HOPS - 1
    g = _k_merge(final, p, g, d if final else d2, b)

  return g


# speedup vs baseline: 1.0521x; 1.0011x over previous
import functools

import jax
import jax.numpy as jnp
from jax import lax
from jax.experimental import pallas as pl
from jax.experimental.pallas import tpu as pltpu
from jax.experimental.pallas import tpu_sc as plsc

N = 10000
D = 128
E = 320000
K_HOPS = 3

NC = 2
NS = 16
NW = NC * NS

EPW = E // NW
CHUNK = 100
NCHUNK = EPW // CHUNK
IBLK = 10
NIBLK = NCHUNK // IBLK
NBUF = 3

ACC0 = 624
ACC15 = 640

NP = 10240
ZROWS = 80


def _mesh():
  return plsc.VectorSubcoreMesh(
      core_axis_name="c", subcore_axis_name="s", num_cores=NC, num_subcores=NS
  )


def _zero_fill(vref, nwords):
  z = jnp.zeros((16,), jnp.float32)

  def body(i, carry):
    vref[pl.ds(i * 16, 16)] = z
    return carry

  lax.fori_loop(0, nwords // 16, body, 0)


def _zero_fill_2d(vref, nrows):
  z = jnp.zeros((16,), jnp.float32)

  def body(r, carry):
    for c in range(D // 16):
      vref[r, pl.ds(c * 16, 16)] = z
    return carry

  lax.fori_loop(0, nrows, body, 0)


def _deg_body(dst_hbm, degp_hbm, idx_v, ones_v, zbuf_v, deg_sh):
  sid = lax.axis_index("s")
  cid = lax.axis_index("c")
  wid = cid * NS + sid

  zslice = NP // NS
  _zero_fill(zbuf_v, zslice)
  pltpu.sync_copy(zbuf_v, deg_sh.at[pl.ds(sid * zslice, zslice)])

  one = jnp.ones((16,), jnp.float32)
  for i in range(8):
    ones_v[pl.ds(i * 16, 16)] = one

  pltpu.sync_copy(dst_hbm.at[wid], idx_v)
  plsc.subcore_barrier()

  def step(j, carry):
    pltpu.sync_copy(ones_v.at[pl.ds(0, CHUNK)], deg_sh.at[idx_v.at[j]],
                    add=True)
    return carry

  lax.fori_loop(0, NCHUNK, step, 0)
  plsc.subcore_barrier()

  pltpu.sync_copy(
      deg_sh.at[pl.ds(sid * zslice, zslice)],
      degp_hbm.at[pl.ds(cid * NP + sid * zslice, zslice)],
  )


def _k_deg(dst3):
  f = pl.kernel(
      _deg_body,
      out_type=jax.ShapeDtypeStruct((NC * NP,), jnp.float32),
      mesh=_mesh(),
      scratch_types=[
          pltpu.VMEM((NCHUNK, CHUNK), jnp.int32),
          pltpu.VMEM((128,), jnp.float32),
          pltpu.VMEM((NP // NS,), jnp.float32),
          pltpu.VMEM_SHARED((NP,), jnp.float32),
      ],
  )
  return f(dst3)


def _hop_body(g_hbm, src_hbm, dst_hbm, p_hbm,
              src_v, dst_v, rows_v, gsem, ssem, acc_sh):
  sid = lax.axis_index("s")
  cid = lax.axis_index("c")
  wid = cid * NS + sid

  _zero_fill_2d(rows_v.at[0], ZROWS)

  @pl.when(sid < NS - 1)
  def _zero_624():
    for z in range(7):
      pltpu.sync_copy(
          rows_v.at[0, pl.ds(0, ZROWS)],
          acc_sh.at[pl.ds(sid * ACC0 + z * ZROWS, ZROWS)],
      )
    pltpu.sync_copy(
        rows_v.at[0, pl.ds(0, ACC0 - 7 * ZROWS)],
        acc_sh.at[pl.ds(sid * ACC0 + 7 * ZROWS, ACC0 - 7 * ZROWS)],
    )

  @pl.when(sid == NS - 1)
  def _zero_640():
    for z in range(8):
      pltpu.sync_copy(
          rows_v.at[0, pl.ds(0, ZROWS)],
          acc_sh.at[pl.ds((NS - 1) * ACC0 + z * ZROWS, ZROWS)],
      )

  def sref(j):
    return src_v.at[(j // IBLK) % 2, j % IBLK]

  def dref(j):
    return dst_v.at[(j // IBLK) % 2, j % IBLK]

  def load_iblk(b):
    pltpu.sync_copy(src_hbm.at[wid, b], src_v.at[b % 2])
    pltpu.sync_copy(dst_hbm.at[wid, b], dst_v.at[b % 2])

  load_iblk(0)
  plsc.subcore_barrier()

  pltpu.async_copy(g_hbm.at[sref(0)], rows_v.at[0], gsem)
  pltpu.async_copy(g_hbm.at[sref(1)], rows_v.at[1], gsem)

  def step(j, carry):
    buf = j % NBUF

    @pl.when(((j + 2) % IBLK == 0) & (j + 2 < NCHUNK))
    def _load_next_iblk():
      load_iblk((j + 2) // IBLK)

    pltpu.make_async_copy(g_hbm.at[sref(j)], rows_v.at[buf], gsem).wait()
    pltpu.async_copy(rows_v.at[buf], acc_sh.at[dref(j)], ssem, add=True)

    @pl.when(j > 0)
    def _drain_prev_scatter():
      pltpu.make_async_copy(
          rows_v.at[(j - 1) % NBUF], acc_sh.at[dref(j - 1)], ssem
      ).wait()

    @pl.when(j + 2 < NCHUNK)
    def _issue_next_gather():
      pltpu.async_copy(g_hbm.at[sref(j + 2)], rows_v.at[(j + 2) % NBUF], gsem)

    return carry

  lax.fori_loop(0, NCHUNK, step, 0)
  pltpu.make_async_copy(
      rows_v.at[(NCHUNK - 1) % NBUF], acc_sh.at[dref(NCHUNK - 1)], ssem
  ).wait()
  plsc.subcore_barrier()

  @pl.when(sid < NS - 1)
  def _write_624():
    pltpu.sync_copy(
        acc_sh.at[pl.ds(sid * ACC0, ACC0)],
        p_hbm.at[cid, pl.ds(sid * ACC0, ACC0)],
    )

  @pl.when(sid == NS - 1)
  def _write_640():
    pltpu.sync_copy(
        acc_sh.at[pl.ds((NS - 1) * ACC0, ACC15)],
        p_hbm.at[cid, pl.ds((NS - 1) * ACC0, ACC15)],
    )


def _k_hop(g, src3, dst3):
  f = pl.kernel(
      _hop_body,
      out_type=jax.ShapeDtypeStruct((NC, N, D), jnp.float32),
      mesh=_mesh(),
      scratch_types=[
          pltpu.VMEM((2, IBLK, CHUNK), jnp.int32),
          pltpu.VMEM((2, IBLK, CHUNK), jnp.int32),
          pltpu.VMEM((NBUF, CHUNK, D), jnp.float32),
          pltpu.SemaphoreType.DMA,
          pltpu.SemaphoreType.DMA,
          pltpu.VMEM_SHARED((N, D), jnp.float32),
      ],
  )
  return f(g, src3, dst3)


def _merge_body(final, p_ref, g_ref, s_ref, b_ref, out_ref):
  v = (p_ref[0] + p_ref[1] + g_ref[...]) * s_ref[:N][:, None]
  if final:
    v = v + b_ref[...][None, :]
  out_ref[...] = v


def _k_merge(final, p, g, scale, b):
  return pl.pallas_call(
      functools.partial(_merge_body, final),
      out_shape=jax.ShapeDtypeStruct((N, D), jnp.float32),
  )(p, g, scale, b)


def _pre_body(x_ref, w_ref, degp_ref, g0_ref, d_ref, d2_ref):
  deg = degp_ref[0, :] + degp_ref[1, :] + 1.0
  d = lax.rsqrt(deg)
  d_ref[...] = d
  d2_ref[...] = 1.0 / deg
  y = lax.dot_general(
      x_ref[...], w_ref[...], (((1,), (1,)), ((), ())),
      preferred_element_type=jnp.float32,
  )
  g0_ref[...] = y * d[:N, None]


def _k_pre(x, W, degp):
  return pl.pallas_call(
      _pre_body,
      out_shape=(
          jax.ShapeDtypeStruct((N, D), jnp.float32),
          jax.ShapeDtypeStruct((NP,), jnp.float32),
          jax.ShapeDtypeStruct((NP,), jnp.float32),
      ),
  )(x, W, degp.reshape(NC, NP))


def kernel(x, edge_index, W, b):
  src = edge_index[0].astype(jnp.int32).reshape(NW, NIBLK, IBLK, CHUNK)
  dst = edge_index[1].astype(jnp.int32).reshape(NW, NIBLK, IBLK, CHUNK)

  degp = _k_deg(edge_index[1].astype(jnp.int32).reshape(NW, NCHUNK, CHUNK))
  g, d, d2 = _k_pre(x, W, degp)

  for k in range(K_HOPS):
    p = _k_hop(g, src, dst)
    final = k == K_HOPS - 1
    g = _k_merge(final, p, g, d if final else d2, b)

  return g
